# baseline (speedup 1.0000x reference)
import functools

import jax
import jax.numpy as jnp
from jax import lax
from jax.experimental import pallas as pl
from jax.experimental.pallas import tpu as pltpu
from jax.experimental.pallas import tpu_sc as plsc

M = 65536
N = 65536
NNZ = 4194304

NC = 2
NS = 16
NW = NC * NS
CHUNK = NNZ // NW
BLK = 4096
NBLK = CHUNK // BLK
W = 16384
FL = 4096
ROWS_PER_SUB = M // NS


def _spmv_body(rows_hbm, cols_hbm, vals_hbm, x_hbm, out_hbm,
               x_v, cols_v0, cols_v1, vals_v0, vals_v1,
               rows_v0, rows_v1, rows_v2, rows_v3,
               prod_v0, prod_v1, win_v, iota_v, acc_sh,
               x_sem, in_sems, rows_sems, scat_sems, misc_sems):
    cols_vs = (cols_v0, cols_v1)
    vals_vs = (vals_v0, vals_v1)
    rows_vs = (rows_v0, rows_v1, rows_v2, rows_v3)
    prod_vs = (prod_v0, prod_v1)
    c = lax.axis_index("c")
    s = lax.axis_index("s")
    wid = s * NC + c
    wbase = wid * CHUNK

    pltpu.async_copy(x_hbm, x_v, x_sem)

    def start_inputs(blk, b, r):
        off = pl.multiple_of(wbase + blk * BLK, BLK)
        pltpu.async_copy(cols_hbm.at[pl.ds(off, BLK)], cols_vs[b],
                         in_sems.at[b, 0])
        pltpu.async_copy(vals_hbm.at[pl.ds(off, BLK)], vals_vs[b],
                         in_sems.at[b, 1])
        pltpu.async_copy(rows_hbm.at[pl.ds(off, BLK)], rows_vs[r],
                         rows_sems.at[r])

    def wait_inputs(b, r):
        pltpu.make_async_copy(cols_hbm.at[pl.ds(0, BLK)], cols_vs[b],
                              in_sems.at[b, 0]).wait()
        pltpu.make_async_copy(vals_hbm.at[pl.ds(0, BLK)], vals_vs[b],
                              in_sems.at[b, 1]).wait()
        pltpu.make_async_copy(rows_hbm.at[pl.ds(0, BLK)], rows_vs[r],
                              rows_sems.at[r]).wait()

    start_inputs(0, 0, 0)
    pltpu.async_copy(rows_hbm.at[pl.ds(wbase, 16)],
                     iota_v.at[pl.ds(0, 16)], misc_sems.at[0])
    pltpu.async_copy(rows_hbm.at[pl.ds(wbase + CHUNK - 16, 16)],
                     iota_v.at[pl.ds(16, 16)], misc_sems.at[1])

    def _zero(i, carry):
        prod_v0[pl.ds(i * 16, 16)] = jnp.zeros((16,), jnp.float32)
        return carry
    lax.fori_loop(0, ROWS_PER_SUB // 16, _zero, 0)
    pltpu.sync_copy(prod_v0, acc_sh.at[pl.ds(s * ROWS_PER_SUB, ROWS_PER_SUB)])

    pltpu.make_async_copy(x_hbm, x_v, x_sem).wait()
    pltpu.make_async_copy(rows_hbm.at[pl.ds(0, 16)],
                          iota_v.at[pl.ds(0, 16)], misc_sems.at[0]).wait()
    pltpu.make_async_copy(rows_hbm.at[pl.ds(0, 16)],
                          iota_v.at[pl.ds(16, 16)], misc_sems.at[1]).wait()
    lo = jnp.min(iota_v[pl.ds(0, 16)])
    hi = jnp.max(iota_v[pl.ds(16, 16)])
    span = hi - lo + 1
    plsc.subcore_barrier()

    @pl.when(span <= W)
    def _fast():
        nfl = (span + FL - 1) // FL

        def _zw(i, carry):
            win_v[pl.ds(i * 16, 16)] = jnp.zeros((16,), jnp.float32)
            return carry
        lax.fori_loop(0, nfl * (FL // 16), _zw, 0)
        stride16 = lax.iota(jnp.int32, 16) * (BLK // 16)

        def _pair(q, carry):
            for b in range(2):
                blk = q * 2 + b

                @pl.when(blk + 1 < NBLK)
                def _():
                    start_inputs(blk + 1, 1 - b, 1 - b)

                wait_inputs(b, b)

                def _inner(i, icarry):
                    idx_e = stride16 + i
                    cols16 = plsc.load_gather(cols_vs[b], [idx_e])
                    vals16 = plsc.load_gather(vals_vs[b], [idx_e])
                    rows16 = plsc.load_gather(rows_vs[b], [idx_e])
                    xg = plsc.load_gather(x_v, [cols16])
                    plsc.addupdate_scatter(win_v, [rows16 - lo],
                                           vals16 * xg)
                    return icarry
                lax.fori_loop(0, BLK // 16, _inner, 0)
            return carry

        lax.fori_loop(0, NBLK // 2, _pair, 0)

        iota16 = lax.iota(jnp.int32, 16)

        def _flush(t, carry):
            fbase = lo + t * FL

            def _bi(k, kcarry):
                idx16 = jnp.minimum(fbase + k * 16 + iota16, M - 1)
                iota_v[pl.ds(k * 16, 16)] = idx16
                return kcarry
            lax.fori_loop(0, FL // 16, _bi, 0)
            pltpu.sync_copy(win_v.at[pl.ds(t * FL, FL)],
                            acc_sh.at[iota_v], add=True)
            return carry

        lax.fori_loop(0, nfl, _flush, 0)

    @pl.when(span > W)
    def _slow():
        def compute(b):
            def _inner(i, icarry):
                cols16 = cols_vs[b][pl.ds(i * 16, 16)]
                vals16 = vals_vs[b][pl.ds(i * 16, 16)]
                xg = plsc.load_gather(x_v, [cols16])
                prod_vs[b][pl.ds(i * 16, 16)] = vals16 * xg
                return icarry
            lax.fori_loop(0, BLK // 16, _inner, 0)

        def fire_scatters(b, r):
            pltpu.async_copy(prod_vs[b], acc_sh.at[rows_vs[r]],
                             scat_sems.at[b], add=True)

        def drain_scatters(b, r):
            pltpu.make_async_copy(prod_vs[b], acc_sh.at[rows_vs[r]],
                                  scat_sems.at[b]).wait()

        def _quad(q, carry):
            for u in range(4):
                blk = q * 4 + u
                b = u % 2
                r = u

                @pl.when(blk + 1 < NBLK)
                def _():
                    start_inputs(blk + 1, 1 - b, (u + 1) % 4)

                wait_inputs(b, r)

                @pl.when(blk >= 2)
                def _():
                    drain_scatters(b, (u + 2) % 4)

                compute(b)
                fire_scatters(b, r)
            return carry

        lax.fori_loop(0, NBLK // 4, _quad, 0)
        drain_scatters(0, 2)
        drain_scatters(1, 3)

    plsc.subcore_barrier()
    pltpu.sync_copy(acc_sh.at[pl.ds(s * ROWS_PER_SUB, ROWS_PER_SUB)],
                    out_hbm.at[c, pl.ds(s * ROWS_PER_SUB, ROWS_PER_SUB)])


_spmv = functools.partial(
    pl.kernel,
    out_type=jax.ShapeDtypeStruct((NC, M), jnp.float32),
    mesh=plsc.VectorSubcoreMesh(core_axis_name="c", subcore_axis_name="s"),
    compiler_params=pltpu.CompilerParams(needs_layout_passes=False),
    scratch_types=[
        pltpu.VMEM((N,), jnp.float32),
        pltpu.VMEM((BLK,), jnp.int32),
        pltpu.VMEM((BLK,), jnp.int32),
        pltpu.VMEM((BLK,), jnp.float32),
        pltpu.VMEM((BLK,), jnp.float32),
        pltpu.VMEM((BLK,), jnp.int32),
        pltpu.VMEM((BLK,), jnp.int32),
        pltpu.VMEM((BLK,), jnp.int32),
        pltpu.VMEM((BLK,), jnp.int32),
        pltpu.VMEM((BLK,), jnp.float32),
        pltpu.VMEM((BLK,), jnp.float32),
        pltpu.VMEM((W,), jnp.float32),
        pltpu.VMEM((FL,), jnp.int32),
        pltpu.VMEM_SHARED((M,), jnp.float32),
        pltpu.SemaphoreType.DMA,
        pltpu.SemaphoreType.DMA((2, 2)),
        pltpu.SemaphoreType.DMA((4,)),
        pltpu.SemaphoreType.DMA((2,)),
        pltpu.SemaphoreType.DMA((2,)),
    ],
)(_spmv_body)


def _finish_body(acc_ref, b_ref, iy_ref, out_ref):
    y = acc_ref[0] + acc_ref[1] - b_ref[...]
    py = y + iy_ref[...] * jnp.maximum(-y, 0.0)
    part_2 = jnp.max(jnp.abs(py))
    part_3 = 1.0 + jnp.max(jnp.abs(b_ref[...]))
    out_ref[0, 0] = part_2 / part_3


_finish = pl.pallas_call(
    _finish_body,
    out_shape=jax.ShapeDtypeStruct((1, 1), jnp.float32),
    out_specs=pl.BlockSpec(memory_space=pltpu.SMEM),
)


def kernel(A_rows, A_cols, A_values, b, c, x, Iy):
    rows1d = A_rows.astype(jnp.int32)
    cols = A_cols.astype(jnp.int32)
    xf = x[:, 0]
    acc = _spmv(rows1d, cols, A_values, xf)
    out = _finish(acc.reshape(NC, 512, 128),
                  b.reshape(512, 128),
                  Iy.reshape(512, 128))
    return out[0, 0]


# speedup vs baseline: 3.3268x; 3.3268x over previous
import functools

import jax
import jax.numpy as jnp
from jax import lax
from jax.experimental import pallas as pl
from jax.experimental.pallas import tpu as pltpu
from jax.experimental.pallas import tpu_sc as plsc

M = 65536
N = 65536
NNZ = 4194304

NC = 2
NS = 16
NW = NC * NS
CHUNK = NNZ // NW
BLK = 2048
NBLK = CHUNK // BLK
WWORDS = 32768
W1 = 4096
FL = 2048
ROWS_PER_SUB = M // NS


def _spmv_body(rows_hbm, cols_hbm, vals_hbm, x_hbm, out_hbm,
               x_v, cols_v0, cols_v1, vals_v0, vals_v1,
               rows_v0, rows_v1, rows_v2, rows_v3,
               prod_v0, prod_v1, win_v, iota_v, acc_sh,
               x_sem, in_sems, rows_sems, scat_sems, misc_sems):
    cols_vs = (cols_v0, cols_v1)
    vals_vs = (vals_v0, vals_v1)
    rows_vs = (rows_v0, rows_v1, rows_v2, rows_v3)
    prod_vs = (prod_v0, prod_v1)
    c = lax.axis_index("c")
    s = lax.axis_index("s")
    wid = s * NC + c
    wbase = wid * CHUNK

    pltpu.async_copy(x_hbm, x_v, x_sem)

    def start_inputs(blk, b, r):
        off = pl.multiple_of(wbase + blk * BLK, BLK)
        pltpu.async_copy(cols_hbm.at[pl.ds(off, BLK)], cols_vs[b],
                         in_sems.at[b, 0])
        pltpu.async_copy(vals_hbm.at[pl.ds(off, BLK)], vals_vs[b],
                         in_sems.at[b, 1])
        pltpu.async_copy(rows_hbm.at[pl.ds(off, BLK)], rows_vs[r],
                         rows_sems.at[r])

    def wait_inputs(b, r):
        pltpu.make_async_copy(cols_hbm.at[pl.ds(0, BLK)], cols_vs[b],
                              in_sems.at[b, 0]).wait()
        pltpu.make_async_copy(vals_hbm.at[pl.ds(0, BLK)], vals_vs[b],
                              in_sems.at[b, 1]).wait()
        pltpu.make_async_copy(rows_hbm.at[pl.ds(0, BLK)], rows_vs[r],
                              rows_sems.at[r]).wait()

    start_inputs(0, 0, 0)
    pltpu.async_copy(rows_hbm.at[pl.ds(wbase, 16)],
                     iota_v.at[pl.ds(0, 16)], misc_sems.at[0])
    pltpu.async_copy(rows_hbm.at[pl.ds(wbase + CHUNK - 16, 16)],
                     iota_v.at[pl.ds(16, 16)], misc_sems.at[1])

    def _zero(i, carry):
        win_v[pl.ds(i * 16, 16)] = jnp.zeros((16,), jnp.float32)
        return carry
    lax.fori_loop(0, ROWS_PER_SUB // 16, _zero, 0)
    pltpu.sync_copy(win_v.at[pl.ds(0, ROWS_PER_SUB)],
                    acc_sh.at[pl.ds(s * ROWS_PER_SUB, ROWS_PER_SUB)])

    pltpu.make_async_copy(x_hbm, x_v, x_sem).wait()
    pltpu.make_async_copy(rows_hbm.at[pl.ds(0, 16)],
                          iota_v.at[pl.ds(0, 16)], misc_sems.at[0]).wait()
    pltpu.make_async_copy(rows_hbm.at[pl.ds(0, 16)],
                          iota_v.at[pl.ds(16, 16)], misc_sems.at[1]).wait()
    lo = jnp.min(iota_v[pl.ds(0, 16)])
    hi = jnp.max(iota_v[pl.ds(16, 16)])
    span = hi - lo + 1
    plsc.subcore_barrier()

    iota16 = lax.iota(jnp.int32, 16)

    def fast_path(S):
        nfl = (span + FL - 1) // FL
        laneS = iota16 % S

        def _zw(i, carry):
            win_v[pl.ds(i * 16, 16)] = jnp.zeros((16,), jnp.float32)
            return carry
        lax.fori_loop(0, nfl * (FL * S // 16), _zw, 0)

        def _pair(q, carry):
            for b in range(2):
                blk = q * 2 + b

                @pl.when(blk + 1 < NBLK)
                def _():
                    start_inputs(blk + 1, 1 - b, 1 - b)

                wait_inputs(b, b)

                @plsc.parallel_loop(0, BLK // 16, 1, unroll=4)
                def _inner(i):
                    cols16 = cols_vs[b][pl.ds(i * 16, 16)]
                    vals16 = vals_vs[b][pl.ds(i * 16, 16)]
                    rows16 = rows_vs[b][pl.ds(i * 16, 16)]
                    xg = plsc.load_gather(x_v, [cols16])
                    widx = (rows16 - lo) * S + laneS
                    plsc.addupdate_scatter(win_v, [widx], vals16 * xg)
            return carry

        lax.fori_loop(0, NBLK // 2, _pair, 0)

        iotaS = iota16 * S

        def _flush(t, carry):
            fbase = lo + t * FL

            def _bi(k, kcarry):
                idx16 = jnp.minimum(fbase + k * 16 + iota16, M - 1)
                iota_v[pl.ds(k * 16, 16)] = idx16
                if S > 1:
                    gidx = (t * FL + k * 16) * S + iotaS
                    tot = plsc.load_gather(win_v, [gidx])
                    for j in range(1, S):
                        tot = tot + plsc.load_gather(win_v, [gidx + j])
                    prod_v0[pl.ds(k * 16, 16)] = tot
                return kcarry
            lax.fori_loop(0, FL // 16, _bi, 0)
            if S > 1:
                pltpu.sync_copy(prod_v0, acc_sh.at[iota_v], add=True)
            else:
                pltpu.sync_copy(win_v.at[pl.ds(t * FL, FL)],
                                acc_sh.at[iota_v], add=True)
            return carry

        lax.fori_loop(0, nfl, _flush, 0)

    @pl.when(span <= W1)
    def _fast8():
        fast_path(8)

    @pl.when(jnp.logical_and(span > W1, span <= WWORDS))
    def _fast1():
        fast_path(1)

    @pl.when(span > WWORDS)
    def _slow():
        def compute(b):
            @plsc.parallel_loop(0, BLK // 16, 1, unroll=4)
            def _inner(i):
                cols16 = cols_vs[b][pl.ds(i * 16, 16)]
                vals16 = vals_vs[b][pl.ds(i * 16, 16)]
                xg = plsc.load_gather(x_v, [cols16])
                prod_vs[b][pl.ds(i * 16, 16)] = vals16 * xg

        def fire_scatters(b, r):
            pltpu.async_copy(prod_vs[b], acc_sh.at[rows_vs[r]],
                             scat_sems.at[b], add=True)

        def drain_scatters(b, r):
            pltpu.make_async_copy(prod_vs[b], acc_sh.at[rows_vs[r]],
                                  scat_sems.at[b]).wait()

        def _quad(q, carry):
            for u in range(4):
                blk = q * 4 + u
                b = u % 2
                r = u

                @pl.when(blk + 1 < NBLK)
                def _():
                    start_inputs(blk + 1, 1 - b, (u + 1) % 4)

                wait_inputs(b, r)

                @pl.when(blk >= 2)
                def _():
                    drain_scatters(b, (u + 2) % 4)

                compute(b)
                fire_scatters(b, r)
            return carry

        lax.fori_loop(0, NBLK // 4, _quad, 0)
        drain_scatters(0, 2)
        drain_scatters(1, 3)

    plsc.subcore_barrier()
    pltpu.sync_copy(acc_sh.at[pl.ds(s * ROWS_PER_SUB, ROWS_PER_SUB)],
                    out_hbm.at[c, pl.ds(s * ROWS_PER_SUB, ROWS_PER_SUB)])


_spmv = functools.partial(
    pl.kernel,
    out_type=jax.ShapeDtypeStruct((NC, M), jnp.float32),
    mesh=plsc.VectorSubcoreMesh(core_axis_name="c", subcore_axis_name="s"),
    compiler_params=pltpu.CompilerParams(needs_layout_passes=False),
    scratch_types=[
        pltpu.VMEM((N,), jnp.float32),
        pltpu.VMEM((BLK,), jnp.int32),
        pltpu.VMEM((BLK,), jnp.int32),
        pltpu.VMEM((BLK,), jnp.float32),
        pltpu.VMEM((BLK,), jnp.float32),
        pltpu.VMEM((BLK,), jnp.int32),
        pltpu.VMEM((BLK,), jnp.int32),
        pltpu.VMEM((BLK,), jnp.int32),
        pltpu.VMEM((BLK,), jnp.int32),
        pltpu.VMEM((BLK,), jnp.float32),
        pltpu.VMEM((BLK,), jnp.float32),
        pltpu.VMEM((WWORDS,), jnp.float32),
        pltpu.VMEM((FL,), jnp.int32),
        pltpu.VMEM_SHARED((M,), jnp.float32),
        pltpu.SemaphoreType.DMA,
        pltpu.SemaphoreType.DMA((2, 2)),
        pltpu.SemaphoreType.DMA((4,)),
        pltpu.SemaphoreType.DMA((2,)),
        pltpu.SemaphoreType.DMA((2,)),
    ],
)(_spmv_body)


def _finish_body(acc_ref, b_ref, iy_ref, out_ref):
    y = acc_ref[0] + acc_ref[1] - b_ref[...]
    py = y + iy_ref[...] * jnp.maximum(-y, 0.0)
    part_2 = jnp.max(jnp.abs(py))
    part_3 = 1.0 + jnp.max(jnp.abs(b_ref[...]))
    out_ref[0, 0] = part_2 / part_3


_finish = pl.pallas_call(
    _finish_body,
    out_shape=jax.ShapeDtypeStruct((1, 1), jnp.float32),
    out_specs=pl.BlockSpec(memory_space=pltpu.SMEM),
)


def kernel(A_rows, A_cols, A_values, b, c, x, Iy):
    rows1d = A_rows.astype(jnp.int32)
    cols = A_cols.astype(jnp.int32)
    xf = x[:, 0]
    acc = _spmv(rows1d, cols, A_values, xf)
    out = _finish(acc.reshape(NC, 512, 128),
                  b.reshape(512, 128),
                  Iy.reshape(512, 128))
    return out[0, 0]


# speedup vs baseline: 3.6998x; 1.1121x over previous
import functools

import jax
import jax.numpy as jnp
from jax import lax
from jax.experimental import pallas as pl
from jax.experimental.pallas import tpu as pltpu
from jax.experimental.pallas import tpu_sc as plsc

M = 65536
N = 65536
NNZ = 4194304

NC = 2
NS = 16
NW = NC * NS
CHUNK = NNZ // NW
BLK = 2048
NBLK = CHUNK // BLK
WWORDS = 28672
W1 = 3584
FL = 2048
ROWS_PER_SUB = M // NS


def _spmv_body(rows_hbm, cols_hbm, vals_hbm, x_hbm, out_hbm,
               x_v, cols_v0, cols_v1, cols_v2, cols_v3,
               vals_v0, vals_v1, vals_v2, vals_v3,
               rows_v0, rows_v1, rows_v2, rows_v3,
               prod_v0, prod_v1, win_v, iota_v, acc_sh,
               x_sem, in_sems, rows_sems, scat_sems, misc_sems):
    cols_vs = (cols_v0, cols_v1, cols_v2, cols_v3)
    vals_vs = (vals_v0, vals_v1, vals_v2, vals_v3)
    rows_vs = (rows_v0, rows_v1, rows_v2, rows_v3)
    prod_vs = (prod_v0, prod_v1)
    c = lax.axis_index("c")
    s = lax.axis_index("s")
    wid = s * NC + c
    wbase = wid * CHUNK

    pltpu.async_copy(x_hbm, x_v, x_sem)

    def start_inputs(blk, b, r):
        off = pl.multiple_of(wbase + blk * BLK, BLK)
        pltpu.async_copy(cols_hbm.at[pl.ds(off, BLK)], cols_vs[b],
                         in_sems.at[b, 0])
        pltpu.async_copy(vals_hbm.at[pl.ds(off, BLK)], vals_vs[b],
                         in_sems.at[b, 1])
        pltpu.async_copy(rows_hbm.at[pl.ds(off, BLK)], rows_vs[r],
                         rows_sems.at[r])

    def wait_inputs(b, r):
        pltpu.make_async_copy(cols_hbm.at[pl.ds(0, BLK)], cols_vs[b],
                              in_sems.at[b, 0]).wait()
        pltpu.make_async_copy(vals_hbm.at[pl.ds(0, BLK)], vals_vs[b],
                              in_sems.at[b, 1]).wait()
        pltpu.make_async_copy(rows_hbm.at[pl.ds(0, BLK)], rows_vs[r],
                              rows_sems.at[r]).wait()

    start_inputs(0, 0, 0)
    pltpu.async_copy(rows_hbm.at[pl.ds(wbase, 16)],
                     iota_v.at[pl.ds(0, 16)], misc_sems.at[0])
    pltpu.async_copy(rows_hbm.at[pl.ds(wbase + CHUNK - 16, 16)],
                     iota_v.at[pl.ds(16, 16)], misc_sems.at[1])

    def _zero(i, carry):
        win_v[pl.ds(i * 16, 16)] = jnp.zeros((16,), jnp.float32)
        return carry
    lax.fori_loop(0, ROWS_PER_SUB // 16, _zero, 0)
    pltpu.sync_copy(win_v.at[pl.ds(0, ROWS_PER_SUB)],
                    acc_sh.at[pl.ds(s * ROWS_PER_SUB, ROWS_PER_SUB)])

    pltpu.make_async_copy(x_hbm, x_v, x_sem).wait()
    pltpu.make_async_copy(rows_hbm.at[pl.ds(0, 16)],
                          iota_v.at[pl.ds(0, 16)], misc_sems.at[0]).wait()
    pltpu.make_async_copy(rows_hbm.at[pl.ds(0, 16)],
                          iota_v.at[pl.ds(16, 16)], misc_sems.at[1]).wait()
    lo = jnp.min(iota_v[pl.ds(0, 16)])
    hi = jnp.max(iota_v[pl.ds(16, 16)])
    span = hi - lo + 1
    plsc.subcore_barrier()

    iota16 = lax.iota(jnp.int32, 16)

    def fast_path(S):
        nfl = (span + FL - 1) // FL
        laneS = iota16 % S

        def _zw(i, carry):
            win_v[pl.ds(i * 16, 16)] = jnp.zeros((16,), jnp.float32)
            return carry
        lax.fori_loop(0, nfl * (FL * S // 16), _zw, 0)

        start_inputs(1, 1, 1)
        start_inputs(2, 2, 2)

        def _quad(q, carry):
            for b in range(4):
                blk = q * 4 + b

                @pl.when(blk + 3 < NBLK)
                def _():
                    start_inputs(blk + 3, (b + 3) % 4, (b + 3) % 4)

                wait_inputs(b, b)

                @plsc.parallel_loop(0, BLK // 16, 1, unroll=4)
                def _inner(i):
                    cols16 = cols_vs[b][pl.ds(i * 16, 16)]
                    vals16 = vals_vs[b][pl.ds(i * 16, 16)]
                    rows16 = rows_vs[b][pl.ds(i * 16, 16)]
                    xg = plsc.load_gather(x_v, [cols16])
                    widx = (rows16 - lo) * S + laneS
                    plsc.addupdate_scatter(win_v, [widx], vals16 * xg)
            return carry

        lax.fori_loop(0, NBLK // 4, _quad, 0)

        iotaS = iota16 * S

        def _flush(t, carry):
            fbase = lo + t * FL

            def _bi(k, kcarry):
                idx16 = jnp.minimum(fbase + k * 16 + iota16, M - 1)
                iota_v[pl.ds(k * 16, 16)] = idx16
                if S > 1:
                    gidx = (t * FL + k * 16) * S + iotaS
                    tot = plsc.load_gather(win_v, [gidx])
                    for j in range(1, S):
                        tot = tot + plsc.load_gather(win_v, [gidx + j])
                    prod_v0[pl.ds(k * 16, 16)] = tot
                return kcarry
            lax.fori_loop(0, FL // 16, _bi, 0)
            if S > 1:
                pltpu.sync_copy(prod_v0, acc_sh.at[iota_v], add=True)
            else:
                pltpu.sync_copy(win_v.at[pl.ds(t * FL, FL)],
                                acc_sh.at[iota_v], add=True)
            return carry

        lax.fori_loop(0, nfl, _flush, 0)

    @pl.when(span <= W1)
    def _fast8():
        fast_path(8)

    @pl.when(jnp.logical_and(span > W1, span <= WWORDS))
    def _fast1():
        fast_path(1)

    @pl.when(span > WWORDS)
    def _slow():
        def compute(b, p):
            @plsc.parallel_loop(0, BLK // 16, 1, unroll=4)
            def _inner(i):
                cols16 = cols_vs[b][pl.ds(i * 16, 16)]
                vals16 = vals_vs[b][pl.ds(i * 16, 16)]
                xg = plsc.load_gather(x_v, [cols16])
                prod_vs[p][pl.ds(i * 16, 16)] = vals16 * xg

        def fire_scatters(b, r):
            pltpu.async_copy(prod_vs[b], acc_sh.at[rows_vs[r]],
                             scat_sems.at[b], add=True)

        def drain_scatters(b, r):
            pltpu.make_async_copy(prod_vs[b], acc_sh.at[rows_vs[r]],
                                  scat_sems.at[b]).wait()

        def _quad(q, carry):
            for u in range(4):
                blk = q * 4 + u
                p = u % 2

                @pl.when(blk + 1 < NBLK)
                def _():
                    start_inputs(blk + 1, (u + 1) % 4, (u + 1) % 4)

                wait_inputs(u, u)

                @pl.when(blk >= 2)
                def _():
                    drain_scatters(p, (u + 2) % 4)

                compute(u, p)
                fire_scatters(p, u)
            return carry

        lax.fori_loop(0, NBLK // 4, _quad, 0)
        drain_scatters(0, 2)
        drain_scatters(1, 3)

    plsc.subcore_barrier()
    pltpu.sync_copy(acc_sh.at[pl.ds(s * ROWS_PER_SUB, ROWS_PER_SUB)],
                    out_hbm.at[c, pl.ds(s * ROWS_PER_SUB, ROWS_PER_SUB)])


_spmv = functools.partial(
    pl.kernel,
    out_type=jax.ShapeDtypeStruct((NC, M), jnp.float32),
    mesh=plsc.VectorSubcoreMesh(core_axis_name="c", subcore_axis_name="s"),
    compiler_params=pltpu.CompilerParams(needs_layout_passes=False),
    scratch_types=[
        pltpu.VMEM((N,), jnp.float32),
        pltpu.VMEM((BLK,), jnp.int32),
        pltpu.VMEM((BLK,), jnp.int32),
        pltpu.VMEM((BLK,), jnp.int32),
        pltpu.VMEM((BLK,), jnp.int32),
        pltpu.VMEM((BLK,), jnp.float32),
        pltpu.VMEM((BLK,), jnp.float32),
        pltpu.VMEM((BLK,), jnp.float32),
        pltpu.VMEM((BLK,), jnp.float32),
        pltpu.VMEM((BLK,), jnp.int32),
        pltpu.VMEM((BLK,), jnp.int32),
        pltpu.VMEM((BLK,), jnp.int32),
        pltpu.VMEM((BLK,), jnp.int32),
        pltpu.VMEM((BLK,), jnp.float32),
        pltpu.VMEM((BLK,), jnp.float32),
        pltpu.VMEM((WWORDS,), jnp.float32),
        pltpu.VMEM((FL,), jnp.int32),
        pltpu.VMEM_SHARED((M,), jnp.float32),
        pltpu.SemaphoreType.DMA,
        pltpu.SemaphoreType.DMA((4, 2)),
        pltpu.SemaphoreType.DMA((4,)),
        pltpu.SemaphoreType.DMA((2,)),
        pltpu.SemaphoreType.DMA((2,)),
    ],
)(_spmv_body)


def _finish_body(acc_ref, b_ref, iy_ref, out_ref):
    y = acc_ref[0] + acc_ref[1] - b_ref[...]
    py = y + iy_ref[...] * jnp.maximum(-y, 0.0)
    part_2 = jnp.max(jnp.abs(py))
    part_3 = 1.0 + jnp.max(jnp.abs(b_ref[...]))
    out_ref[0, 0] = part_2 / part_3


_finish = pl.pallas_call(
    _finish_body,
    out_shape=jax.ShapeDtypeStruct((1, 1), jnp.float32),
    out_specs=pl.BlockSpec(memory_space=pltpu.SMEM),
)


def kernel(A_rows, A_cols, A_values, b, c, x, Iy):
    rows1d = A_rows.astype(jnp.int32)
    cols = A_cols.astype(jnp.int32)
    xf = x[:, 0]
    acc = _spmv(rows1d, cols, A_values, xf)
    out = _finish(acc.reshape(NC, 512, 128),
                  b.reshape(512, 128),
                  Iy.reshape(512, 128))
    return out[0, 0]


# speedup vs baseline: 3.9279x; 1.0616x over previous
import functools

import jax
import jax.numpy as jnp
from jax import lax
from jax.experimental import pallas as pl
from jax.experimental.pallas import tpu as pltpu
from jax.experimental.pallas import tpu_sc as plsc

M = 65536
N = 65536
NNZ = 4194304

NC = 2
NS = 16
NW = NC * NS
CHUNK = NNZ // NW
BLK = 2048
NBLK = CHUNK // BLK
WWORDS = 28672
W1 = 3584
FL = 2048
ROWS_PER_SUB = M // NS


def _spmv_body(rows_hbm, cols_hbm, vals_hbm, x_hbm, out_hbm,
               x_v, cols_v0, cols_v1, cols_v2, cols_v3,
               vals_v0, vals_v1, vals_v2, vals_v3,
               rows_v0, rows_v1, rows_v2, rows_v3,
               prod_v0, prod_v1, win_v, iota_v, iota5_v, acc_sh,
               x_sem, in_sems, rows_sems, scat_sems, misc_sems):
    cols_vs = (cols_v0, cols_v1, cols_v2, cols_v3)
    vals_vs = (vals_v0, vals_v1, vals_v2, vals_v3)
    rows_vs = (rows_v0, rows_v1, rows_v2, rows_v3)
    prod_vs = (prod_v0, prod_v1)
    c = lax.axis_index("c")
    s = lax.axis_index("s")
    wid = s * NC + c
    wbase = wid * CHUNK

    pltpu.async_copy(x_hbm, x_v, x_sem)

    def start_inputs(blk, b, r):
        off = pl.multiple_of(wbase + blk * BLK, BLK)
        pltpu.async_copy(cols_hbm.at[pl.ds(off, BLK)], cols_vs[b],
                         in_sems.at[b, 0])
        pltpu.async_copy(vals_hbm.at[pl.ds(off, BLK)], vals_vs[b],
                         in_sems.at[b, 1])
        pltpu.async_copy(rows_hbm.at[pl.ds(off, BLK)], rows_vs[r],
                         rows_sems.at[r])

    def wait_inputs(b, r):
        pltpu.make_async_copy(cols_hbm.at[pl.ds(0, BLK)], cols_vs[b],
                              in_sems.at[b, 0]).wait()
        pltpu.make_async_copy(vals_hbm.at[pl.ds(0, BLK)], vals_vs[b],
                              in_sems.at[b, 1]).wait()
        pltpu.make_async_copy(rows_hbm.at[pl.ds(0, BLK)], rows_vs[r],
                              rows_sems.at[r]).wait()

    start_inputs(0, 0, 0)
    pltpu.async_copy(rows_hbm.at[pl.ds(wbase, 16)],
                     iota_v.at[pl.ds(0, 16)], misc_sems.at[0])
    pltpu.async_copy(rows_hbm.at[pl.ds(wbase + CHUNK - 16, 16)],
                     iota_v.at[pl.ds(16, 16)], misc_sems.at[1])

    def _zero(i, carry):
        win_v[pl.ds(i * 16, 16)] = jnp.zeros((16,), jnp.float32)
        return carry
    lax.fori_loop(0, ROWS_PER_SUB // 16, _zero, 0)
    pltpu.sync_copy(win_v.at[pl.ds(0, ROWS_PER_SUB)],
                    acc_sh.at[pl.ds(s * ROWS_PER_SUB, ROWS_PER_SUB)])

    pltpu.make_async_copy(x_hbm, x_v, x_sem).wait()
    pltpu.make_async_copy(rows_hbm.at[pl.ds(0, 16)],
                          iota_v.at[pl.ds(0, 16)], misc_sems.at[0]).wait()
    pltpu.make_async_copy(rows_hbm.at[pl.ds(0, 16)],
                          iota_v.at[pl.ds(16, 16)], misc_sems.at[1]).wait()
    lo = jnp.min(iota_v[pl.ds(0, 16)])
    hi = jnp.max(iota_v[pl.ds(16, 16)])
    span = hi - lo + 1
    plsc.subcore_barrier()

    iota16 = lax.iota(jnp.int32, 16)

    def fast_path(S, FLT, iota_ref):
        nfl = (span + FLT - 1) // FLT
        laneS = iota16 % S

        def _zw(i, carry):
            win_v[pl.ds(i * 16, 16)] = jnp.zeros((16,), jnp.float32)
            return carry
        lax.fori_loop(0, nfl * (FLT * S // 16), _zw, 0)

        start_inputs(1, 1, 1)
        start_inputs(2, 2, 2)

        def _quad(q, carry):
            for b in range(4):
                blk = q * 4 + b

                @pl.when(blk + 3 < NBLK)
                def _():
                    start_inputs(blk + 3, (b + 3) % 4, (b + 3) % 4)

                wait_inputs(b, b)

                @plsc.parallel_loop(0, BLK // 16, 1, unroll=4)
                def _inner(i):
                    cols16 = cols_vs[b][pl.ds(i * 16, 16)]
                    vals16 = vals_vs[b][pl.ds(i * 16, 16)]
                    rows16 = rows_vs[b][pl.ds(i * 16, 16)]
                    xg = plsc.load_gather(x_v, [cols16])
                    widx = (rows16 - lo) * S + laneS
                    plsc.addupdate_scatter(win_v, [widx], vals16 * xg)
            return carry

        lax.fori_loop(0, NBLK // 4, _quad, 0)

        iotaS = iota16 * S

        def _flush(t, carry):
            fbase = lo + t * FLT

            def _bi(k, kcarry):
                idx16 = jnp.minimum(fbase + k * 16 + iota16, M - 1)
                iota_ref[pl.ds(k * 16, 16)] = idx16
                if S > 1:
                    gidx = (t * FLT + k * 16) * S + iotaS
                    tot = plsc.load_gather(win_v, [gidx])
                    for j in range(1, S):
                        tot = tot + plsc.load_gather(win_v, [gidx + j])
                    prod_v0[pl.ds(k * 16, 16)] = tot
                return kcarry
            lax.fori_loop(0, FLT // 16, _bi, 0)
            if S > 1:
                pltpu.sync_copy(prod_v0.at[pl.ds(0, FLT)],
                                acc_sh.at[iota_ref], add=True)
            else:
                pltpu.sync_copy(win_v.at[pl.ds(t * FLT, FLT)],
                                acc_sh.at[iota_ref], add=True)
            return carry

        lax.fori_loop(0, nfl, _flush, 0)

    @pl.when(span <= W1)
    def _fast8():
        fast_path(8, 512, iota5_v)

    @pl.when(jnp.logical_and(span > W1, span <= WWORDS))
    def _fast1():
        fast_path(1, FL, iota_v)

    @pl.when(span > WWORDS)
    def _slow():
        def compute(b, p):
            @plsc.parallel_loop(0, BLK // 16, 1, unroll=4)
            def _inner(i):
                cols16 = cols_vs[b][pl.ds(i * 16, 16)]
                vals16 = vals_vs[b][pl.ds(i * 16, 16)]
                xg = plsc.load_gather(x_v, [cols16])
                prod_vs[p][pl.ds(i * 16, 16)] = vals16 * xg

        def fire_scatters(b, r):
            pltpu.async_copy(prod_vs[b], acc_sh.at[rows_vs[r]],
                             scat_sems.at[b], add=True)

        def drain_scatters(b, r):
            pltpu.make_async_copy(prod_vs[b], acc_sh.at[rows_vs[r]],
                                  scat_sems.at[b]).wait()

        def _quad(q, carry):
            for u in range(4):
                blk = q * 4 + u
                p = u % 2

                @pl.when(blk + 1 < NBLK)
                def _():
                    start_inputs(blk + 1, (u + 1) % 4, (u + 1) % 4)

                wait_inputs(u, u)

                @pl.when(blk >= 2)
                def _():
                    drain_scatters(p, (u + 2) % 4)

                compute(u, p)
                fire_scatters(p, u)
            return carry

        lax.fori_loop(0, NBLK // 4, _quad, 0)
        drain_scatters(0, 2)
        drain_scatters(1, 3)

    plsc.subcore_barrier()
    pltpu.sync_copy(acc_sh.at[pl.ds(s * ROWS_PER_SUB, ROWS_PER_SUB)],
                    out_hbm.at[c, pl.ds(s * ROWS_PER_SUB, ROWS_PER_SUB)])


_spmv = functools.partial(
    pl.kernel,
    out_type=jax.ShapeDtypeStruct((NC, M), jnp.float32),
    mesh=plsc.VectorSubcoreMesh(core_axis_name="c", subcore_axis_name="s"),
    compiler_params=pltpu.CompilerParams(needs_layout_passes=False),
    scratch_types=[
        pltpu.VMEM((N,), jnp.float32),
        pltpu.VMEM((BLK,), jnp.int32),
        pltpu.VMEM((BLK,), jnp.int32),
        pltpu.VMEM((BLK,), jnp.int32),
        pltpu.VMEM((BLK,), jnp.int32),
        pltpu.VMEM((BLK,), jnp.float32),
        pltpu.VMEM((BLK,), jnp.float32),
        pltpu.VMEM((BLK,), jnp.float32),
        pltpu.VMEM((BLK,), jnp.float32),
        pltpu.VMEM((BLK,), jnp.int32),
        pltpu.VMEM((BLK,), jnp.int32),
        pltpu.VMEM((BLK,), jnp.int32),
        pltpu.VMEM((BLK,), jnp.int32),
        pltpu.VMEM((BLK,), jnp.float32),
        pltpu.VMEM((BLK,), jnp.float32),
        pltpu.VMEM((WWORDS,), jnp.float32),
        pltpu.VMEM((FL,), jnp.int32),
        pltpu.VMEM((512,), jnp.int32),
        pltpu.VMEM_SHARED((M,), jnp.float32),
        pltpu.SemaphoreType.DMA,
        pltpu.SemaphoreType.DMA((4, 2)),
        pltpu.SemaphoreType.DMA((4,)),
        pltpu.SemaphoreType.DMA((2,)),
        pltpu.SemaphoreType.DMA((2,)),
    ],
)(_spmv_body)


def _finish_body(acc_ref, b_ref, iy_ref, out_ref):
    y = acc_ref[0] + acc_ref[1] - b_ref[...]
    py = y + iy_ref[...] * jnp.maximum(-y, 0.0)
    part_2 = jnp.max(jnp.abs(py))
    part_3 = 1.0 + jnp.max(jnp.abs(b_ref[...]))
    out_ref[0, 0] = part_2 / part_3


_finish = pl.pallas_call(
    _finish_body,
    out_shape=jax.ShapeDtypeStruct((1, 1), jnp.float32),
    out_specs=pl.BlockSpec(memory_space=pltpu.SMEM),
)


def kernel(A_rows, A_cols, A_values, b, c, x, Iy):
    rows1d = A_rows.astype(jnp.int32)
    cols = A_cols.astype(jnp.int32)
    xf = x[:, 0]
    acc = _spmv(rows1d, cols, A_values, xf)
    out = _finish(acc.reshape(NC, 512, 128),
                  b.reshape(512, 128),
                  Iy.reshape(512, 128))
    return out[0, 0]
